# baseline (speedup 1.0000x reference)
import functools

import jax
import jax.numpy as jnp
from jax import lax
from jax.experimental import pallas as pl
from jax.experimental.pallas import tpu as pltpu
from jax.experimental.pallas import tpu_sc as plsc

B, S = 4, 2048
E = 192
H = 4 * E
N = B * S
EPS = 1e-12
DESC, TGT = 2, 3
P4 = E // 4
P3 = E // 3
N_TT = 8
PROP_ROWS = 40
VOCAB_ROWS = 100000

_NC, _NS = 2, 16
_NW = _NC * _NS
NH = N
_TPW = NH // _NW
_CH = 128
_NCH = _TPW // _CH


LW = 128


def _sc_gather_body(ids_hbm, tab_hbm, out_hbm, idx_v, rows_v, sem):
    w = lax.axis_index("s") * _NC + lax.axis_index("c")
    pltpu.sync_copy(ids_hbm.at[pl.ds(w * _NCH, _NCH)], idx_v)
    copies = [
        pltpu.async_copy(tab_hbm.at[idx_v.at[j]],
                         rows_v.at[pl.ds(j * _CH, _CH)], sem)
        for j in range(_NCH)
    ]
    for c in copies:
        c.wait()
    pltpu.sync_copy(rows_v, out_hbm.at[pl.ds(w * _TPW, _TPW)])


@functools.lru_cache(maxsize=1)
def _sc_gather():
    return pl.kernel(
        _sc_gather_body,
        mesh=plsc.VectorSubcoreMesh(core_axis_name="c", subcore_axis_name="s"),
        compiler_params=pltpu.CompilerParams(use_tc_tiling_on_sc=True),
        out_type=jax.ShapeDtypeStruct((NH, LW), jnp.float32),
        scratch_types=[
            pltpu.VMEM((_NCH, _CH), jnp.int32),
            pltpu.VMEM((_TPW, LW), jnp.float32),
            pltpu.SemaphoreType.DMA,
        ],
    )

PACK_C = 8192


def _pack_body(w_ref, o_ref):
    wu = lax.bitcast_convert_type(w_ref[...], jnp.uint32)
    hi = jnp.concatenate(
        [wu[LW:] & jnp.uint32(0xFFFF0000),
         jnp.zeros((2 * LW - E, PACK_C), jnp.uint32)], axis=0)
    packed = (wu[:LW] >> 16) | hi
    o_ref[...] = lax.bitcast_convert_type(packed.T, jnp.float32)


_pack_call = functools.partial(
    pl.pallas_call, _pack_body,
    grid=(-(-VOCAB_ROWS // PACK_C),),
    in_specs=[pl.BlockSpec((E, PACK_C), lambda i: (0, i))],
    out_specs=pl.BlockSpec((PACK_C, LW), lambda i: (i, 0)),
    out_shape=jax.ShapeDtypeStruct((VOCAB_ROWS, LW), jnp.float32),
)

BLK = 1024
G = N // BLK
_PROWS = (3, 4, 9, 5, 3, 3, 7)
CTAB_ROWS = 48


_CONTRACT0 = (((0,), (0,)), ((), ()))


def _tc_body(xp_ref, pe_ref, ints_ref, flts_ref, ttab_ref,
             g_ref, b_ref, o_ref):
    xi = lax.bitcast_convert_type(xp_ref[...], jnp.uint32)
    lo = lax.bitcast_convert_type(xi << 16, jnp.float32)
    hi = lax.bitcast_convert_type(xi & jnp.uint32(0xFFFF0000),
                                  jnp.float32)
    x = jnp.concatenate([lo, hi[:, :E - LW]], axis=1)
    pe = pe_ref[...]
    ints = ints_ref[...]
    flts = flts_ref[...]

    oh = [(ints[0:1, :] == lax.broadcasted_iota(jnp.int32, (N_TT, BLK), 0)
           ).astype(jnp.float32)]
    for k, nk in enumerate(_PROWS):
        oh.append((ints[k + 1:k + 2, :]
                   == lax.broadcasted_iota(jnp.int32, (nk, BLK), 0)
                   ).astype(jnp.float32))
    oh.append(jnp.zeros((CTAB_ROWS - N_TT - sum(_PROWS), BLK), jnp.float32))
    acc = jnp.concatenate(oh, axis=0)
    tp = lax.dot_general(acc, ttab_ref[...], _CONTRACT0,
                         preferred_element_type=jnp.float32)

    cols = lax.dot_general(flts, jnp.eye(4, dtype=jnp.float32), _CONTRACT0,
                           preferred_element_type=jnp.float32)
    md = cols[:, 0:1]
    tv = cols[:, 1:2]
    ttc = cols[:, 2:3]
    scale = (1.0 + jnp.where(ttc == float(DESC), md, 0.0)
                 + jnp.where(ttc == float(TGT), tv, 0.0))
    xs = x * scale

    emb = jnp.concatenate([xs, pe, tp], axis=1)
    mu = jnp.mean(emb, axis=1, keepdims=True)
    m2 = jnp.mean(emb * emb, axis=1, keepdims=True)
    var = m2 - mu * mu
    o_ref[...] = (emb - mu) * lax.rsqrt(var + EPS) * g_ref[...] + b_ref[...]


_TC_KW = dict(
    grid=(G,),
    in_specs=[
        pl.BlockSpec((BLK, LW), lambda i: (i, 0)),
        pl.BlockSpec((BLK, E), lambda i: (i, 0)),
        pl.BlockSpec((8, BLK), lambda i: (0, i)),
        pl.BlockSpec((4, BLK), lambda i: (0, i)),
        pl.BlockSpec((CTAB_ROWS, 2 * E), lambda i: (0, 0)),
        pl.BlockSpec((1, H), lambda i: (0, 0)),
        pl.BlockSpec((1, H), lambda i: (0, 0)),
    ],
    out_specs=pl.BlockSpec((BLK, H), lambda i: (i, 0)),
    out_shape=jax.ShapeDtypeStruct((N, H), jnp.float32),
)

_tc_call = functools.partial(pl.pallas_call, _tc_body, **_TC_KW)


def kernel(input_ids, token_type_ids, pos_embeds, pos_embeds_shape,
           atom_props, bond_props, mol_desc, target_values, word_emb,
           type_emb, in_ring_emb, charge_emb, hyb_emb, chi_emb, arom_emb,
           conj_emb, stereo_emb, ln_gamma, ln_beta):
    ids2 = input_ids.reshape(_NW * _NCH, _CH)
    tab_p = _pack_call()(word_emb.T)
    xp = _sc_gather()(ids2, tab_p)
    pe = pos_embeds.reshape(N, E)

    ints = jnp.concatenate(
        [token_type_ids.reshape(1, N),
         atom_props.reshape(N, 4).T,
         bond_props.reshape(N, 3).T], axis=0)
    flts = jnp.concatenate(
        [mol_desc.reshape(1, N), target_values.reshape(1, N),
         token_type_ids.astype(jnp.float32).reshape(1, N),
         jnp.zeros((1, N), jnp.float32)], axis=0)

    ctab = jnp.zeros((CTAB_ROWS, 2 * E), jnp.float32)
    ctab = ctab.at[0:8, 0:E].set(type_emb)
    ctab = ctab.at[8:11, E + 0 * P4:E + 1 * P4].set(in_ring_emb)
    ctab = ctab.at[11:15, E + 1 * P4:E + 2 * P4].set(charge_emb)
    ctab = ctab.at[15:24, E + 2 * P4:E + 3 * P4].set(hyb_emb)
    ctab = ctab.at[24:29, E + 3 * P4:E + 4 * P4].set(chi_emb)
    ctab = ctab.at[29:32, E + 0 * P3:E + 1 * P3].set(arom_emb)
    ctab = ctab.at[32:35, E + 1 * P3:E + 2 * P3].set(conj_emb)
    ctab = ctab.at[35:42, E + 2 * P3:E + 3 * P3].set(stereo_emb)

    out = _tc_call()(xp, pe, ints, flts, ctab,
                     ln_gamma.reshape(1, H), ln_beta.reshape(1, H))
    return out.reshape(B, S, H)


# speedup vs baseline: 1.0236x; 1.0236x over previous
import functools

import jax
import jax.numpy as jnp
from jax import lax
from jax.experimental import pallas as pl
from jax.experimental.pallas import tpu as pltpu
from jax.experimental.pallas import tpu_sc as plsc

B, S = 4, 2048
E = 192
H = 4 * E
N = B * S
EPS = 1e-12
DESC, TGT = 2, 3
P4 = E // 4
P3 = E // 3
N_TT = 8
PROP_ROWS = 40
VOCAB_ROWS = 100000

_NC, _NS = 2, 16
_NW = _NC * _NS
NH = N
_TPW = NH // _NW
_CH = 128
_NCH = _TPW // _CH


LW = 128


def _sc_gather_body(ids_hbm, tab_hbm, out_hbm, idx_v, rows_v, sem):
    w = lax.axis_index("s") * _NC + lax.axis_index("c")
    pltpu.sync_copy(ids_hbm.at[pl.ds(w * _NCH, _NCH)], idx_v)
    copies = [
        pltpu.async_copy(tab_hbm.at[idx_v.at[j]],
                         rows_v.at[pl.ds(j * _CH, _CH)], sem)
        for j in range(_NCH)
    ]
    for c in copies:
        c.wait()
    pltpu.sync_copy(rows_v, out_hbm.at[pl.ds(w * _TPW, _TPW)])


@functools.lru_cache(maxsize=1)
def _sc_gather():
    return pl.kernel(
        _sc_gather_body,
        mesh=plsc.VectorSubcoreMesh(core_axis_name="c", subcore_axis_name="s"),
        compiler_params=pltpu.CompilerParams(use_tc_tiling_on_sc=True),
        out_type=jax.ShapeDtypeStruct((NH, LW), jnp.float32),
        scratch_types=[
            pltpu.VMEM((_NCH, _CH), jnp.int32),
            pltpu.VMEM((_TPW, LW), jnp.float32),
            pltpu.SemaphoreType.DMA,
        ],
    )

PACK_C = 16384


def _pack_body(w_ref, o_ref):
    wu = lax.bitcast_convert_type(w_ref[...], jnp.uint32)
    hi = jnp.concatenate(
        [wu[LW:] & jnp.uint32(0xFFFF0000),
         jnp.zeros((2 * LW - E, PACK_C), jnp.uint32)], axis=0)
    packed = (wu[:LW] >> 16) | hi
    o_ref[...] = lax.bitcast_convert_type(packed.T, jnp.float32)


_pack_call = functools.partial(
    pl.pallas_call, _pack_body,
    grid=(-(-VOCAB_ROWS // PACK_C),),
    in_specs=[pl.BlockSpec((E, PACK_C), lambda i: (0, i))],
    out_specs=pl.BlockSpec((PACK_C, LW), lambda i: (i, 0)),
    out_shape=jax.ShapeDtypeStruct((VOCAB_ROWS, LW), jnp.float32),
)

BLK = 2048
G = N // BLK
_PROWS = (3, 4, 9, 5, 3, 3, 7)
CTAB_ROWS = 48


_CONTRACT0 = (((0,), (0,)), ((), ()))


def _tc_body(xp_ref, pe_ref, ints_ref, flts_ref, ttab_ref,
             g_ref, b_ref, o_ref):
    xi = lax.bitcast_convert_type(xp_ref[...], jnp.uint32)
    lo = lax.bitcast_convert_type(xi << 16, jnp.float32)
    hi = lax.bitcast_convert_type(xi & jnp.uint32(0xFFFF0000),
                                  jnp.float32)
    x = jnp.concatenate([lo, hi[:, :E - LW]], axis=1)
    pe = pe_ref[...]
    ints = ints_ref[...]
    flts = flts_ref[...]

    oh = [(ints[0:1, :] == lax.broadcasted_iota(jnp.int32, (N_TT, BLK), 0)
           ).astype(jnp.float32)]
    for k, nk in enumerate(_PROWS):
        oh.append((ints[k + 1:k + 2, :]
                   == lax.broadcasted_iota(jnp.int32, (nk, BLK), 0)
                   ).astype(jnp.float32))
    oh.append(jnp.zeros((CTAB_ROWS - N_TT - sum(_PROWS), BLK), jnp.float32))
    acc = jnp.concatenate(oh, axis=0)
    tp = lax.dot_general(acc, ttab_ref[...], _CONTRACT0,
                         preferred_element_type=jnp.float32)

    cols = lax.dot_general(flts, jnp.eye(4, dtype=jnp.float32), _CONTRACT0,
                           preferred_element_type=jnp.float32)
    md = cols[:, 0:1]
    tv = cols[:, 1:2]
    ttc = cols[:, 2:3]
    scale = (1.0 + jnp.where(ttc == float(DESC), md, 0.0)
                 + jnp.where(ttc == float(TGT), tv, 0.0))
    xs = x * scale

    emb = jnp.concatenate([xs, pe, tp], axis=1)
    mu = jnp.mean(emb, axis=1, keepdims=True)
    m2 = jnp.mean(emb * emb, axis=1, keepdims=True)
    var = m2 - mu * mu
    o_ref[...] = (emb - mu) * lax.rsqrt(var + EPS) * g_ref[...] + b_ref[...]


_TC_KW = dict(
    grid=(G,),
    in_specs=[
        pl.BlockSpec((BLK, LW), lambda i: (i, 0)),
        pl.BlockSpec((BLK, E), lambda i: (i, 0)),
        pl.BlockSpec((8, BLK), lambda i: (0, i)),
        pl.BlockSpec((4, BLK), lambda i: (0, i)),
        pl.BlockSpec((CTAB_ROWS, 2 * E), lambda i: (0, 0)),
        pl.BlockSpec((1, H), lambda i: (0, 0)),
        pl.BlockSpec((1, H), lambda i: (0, 0)),
    ],
    out_specs=pl.BlockSpec((BLK, H), lambda i: (i, 0)),
    out_shape=jax.ShapeDtypeStruct((N, H), jnp.float32),
)

_tc_call = functools.partial(pl.pallas_call, _tc_body, **_TC_KW)


def kernel(input_ids, token_type_ids, pos_embeds, pos_embeds_shape,
           atom_props, bond_props, mol_desc, target_values, word_emb,
           type_emb, in_ring_emb, charge_emb, hyb_emb, chi_emb, arom_emb,
           conj_emb, stereo_emb, ln_gamma, ln_beta):
    ids2 = input_ids.reshape(_NW * _NCH, _CH)
    tab_p = _pack_call()(word_emb.T)
    xp = _sc_gather()(ids2, tab_p)
    pe = pos_embeds.reshape(N, E)

    ints = jnp.concatenate(
        [token_type_ids.reshape(1, N),
         atom_props.reshape(N, 4).T,
         bond_props.reshape(N, 3).T], axis=0)
    flts = jnp.concatenate(
        [mol_desc.reshape(1, N), target_values.reshape(1, N),
         token_type_ids.astype(jnp.float32).reshape(1, N),
         jnp.zeros((1, N), jnp.float32)], axis=0)

    ctab = jnp.zeros((CTAB_ROWS, 2 * E), jnp.float32)
    ctab = ctab.at[0:8, 0:E].set(type_emb)
    ctab = ctab.at[8:11, E + 0 * P4:E + 1 * P4].set(in_ring_emb)
    ctab = ctab.at[11:15, E + 1 * P4:E + 2 * P4].set(charge_emb)
    ctab = ctab.at[15:24, E + 2 * P4:E + 3 * P4].set(hyb_emb)
    ctab = ctab.at[24:29, E + 3 * P4:E + 4 * P4].set(chi_emb)
    ctab = ctab.at[29:32, E + 0 * P3:E + 1 * P3].set(arom_emb)
    ctab = ctab.at[32:35, E + 1 * P3:E + 2 * P3].set(conj_emb)
    ctab = ctab.at[35:42, E + 2 * P3:E + 3 * P3].set(stereo_emb)

    out = _tc_call()(xp, pe, ints, flts, ctab,
                     ln_gamma.reshape(1, H), ln_beta.reshape(1, H))
    return out.reshape(B, S, H)
